# baseline (speedup 1.0000x reference)
import math
from functools import partial

import jax
import jax.numpy as jnp
from jax.experimental import pallas as pl
from jax.experimental.pallas import tpu as pltpu

PI = math.pi
CHUNK = 512
SUB = 256
VP = 128


def _gelu(z):
    return 0.5 * z * (1.0 + jax.lax.erf(z * (1.0 / math.sqrt(2.0))))


def _cossin(x):
    nf = jnp.round(x * (2.0 / math.pi))
    ni = nf.astype(jnp.int32)
    r = x - nf * 1.5707963705062866
    r = r - nf * (-4.371139000186245e-08)
    u = r * r
    c = (1.0 / 24.0) - u * (1.0 / 720.0)
    c = c * u - 0.5
    c = c * u + 1.0
    s = (1.0 / 120.0) - u * (1.0 / 5040.0)
    s = s * u - (1.0 / 6.0)
    s = s * u + 1.0
    s = s * r
    b0 = (ni & 1) != 0
    b1 = (ni & 2) != 0
    cos_base = jnp.where(b0, s, c)
    sin_base = jnp.where(b0, c, s)
    cosx = jnp.where(b0 ^ b1, -cos_base, cos_base)
    sinx = jnp.where(b1, -sin_base, sin_base)
    return cosx, sinx


def _trig_kernel(phi_ref, cos_ref, sin_ref):
    p = phi_ref[...]
    cp, sp = _cossin(p)
    cos_ref[...] = cp
    sin_ref[...] = sp


def _phase_tables(pos_phases):
    L, D = pos_phases.shape
    tc = 512
    return pl.pallas_call(
        _trig_kernel,
        grid=(L // tc,),
        in_specs=[pl.BlockSpec((tc, D), lambda i: (i, 0))],
        out_specs=[pl.BlockSpec((tc, D), lambda i: (i, 0)),
                   pl.BlockSpec((tc, D), lambda i: (i, 0))],
        out_shape=[jax.ShapeDtypeStruct((L, D), jnp.float32),
                   jax.ShapeDtypeStruct((L, D), jnp.float32)],
        compiler_params=pltpu.CompilerParams(
            dimension_semantics=("parallel",)),
        name="phasor_trig_tables",
    )(pos_phases)


def _block_kernel(x_ref, cphi_ref, sphi_ref, ms_ref,
                  wcat_ref, bcat_ref,
                  wo_ref, bo_ref, ws1_ref, bs1_ref, ws2_ref, bs2_ref,
                  wkv_ref, bkv_ref, lng_ref, lnb_ref,
                  wt1_ref, bt1_ref, wt2_ref, bt2_ref,
                  out_ref, carry_ref, state_ref):
    C = x_ref.shape[1]
    D = x_ref.shape[2]
    S = SUB
    i = pl.program_id(1)

    @pl.when(i == 0)
    def _():
        carry_ref[...] = jnp.zeros_like(carry_ref)
        state_ref[...] = jnp.zeros_like(state_ref)

    ms = jnp.abs(ms_ref[...])
    f32 = jnp.float32
    dot = partial(jnp.dot, preferred_element_type=f32)

    rows = jax.lax.broadcasted_iota(jnp.int32, (S, S), 0)
    cols = jax.lax.broadcasted_iota(jnp.int32, (S, S), 1)
    tri = (rows >= cols).astype(f32)

    carry = carry_ref[...]
    state = state_ref[...]

    for k in range(C // S):
        sl = slice(k * S, (k + 1) * S)
        x = x_ref[0, sl, :]
        cphi = cphi_ref[sl, :]
        sphi = sphi_ref[sl, :]

        z = dot(x, wcat_ref[...]) + bcat_ref[...]
        v1 = z[:, :D]
        mag = jax.nn.sigmoid(z[:, D:2 * D]) * ms
        dq = z[:, 2 * D:3 * D]
        qp = jnp.tanh(z[:, 3 * D:3 * D + VP]) * PI
        vals = z[:, 3 * D + VP:3 * D + 2 * VP]
        gate = jax.nn.sigmoid(z[:, 3 * D + 2 * VP:3 * D + 2 * VP + 1])
        wv1 = mag * v1
        gate128 = jnp.broadcast_to(gate, (S, VP))

        big = jnp.concatenate(
            [wv1 * cphi, wv1 * sphi, mag, x, gate128], axis=1)
        cums = dot(tri, big) + carry
        carry = cums[S - 1:S, :]
        cum_c = cums[:, :D]
        cum_s = cums[:, D:2 * D]
        cum_m = cums[:, 2 * D:3 * D]
        cum_x = cums[:, 3 * D:4 * D]
        cum_g = cums[:, 4 * D:]

        inv_sq = jax.lax.rsqrt(cum_m + 1e-8)
        cdq, sdq = _cossin(dq)
        cpq = cphi * cdq - sphi * sdq
        spq = sphi * cdq + cphi * sdq
        pos_ret = (cum_c * cpq + cum_s * spq) * inv_sq * (1.0 / math.sqrt(D))
        pos_out = dot(pos_ret, wo_ref[...]) + bo_ref[...]

        posf = (jax.lax.broadcasted_iota(jnp.int32, (S, 1), 0)
                + (i * C + k * S + 1)).astype(f32)
        ctx = cum_x * (1.0 / posf)
        cat = jnp.concatenate([x, ctx], axis=1)
        h = _gelu(dot(cat, ws1_ref[...]) + bs1_ref[...])
        sp = jnp.tanh(dot(h, ws2_ref[...]) + bs2_ref[...]) * PI
        cqp, sqp = _cossin(qp)
        csp, ssp = _cossin(sp)
        q_feat = jnp.concatenate([cqp, sqp], axis=1)
        k_feat = jnp.concatenate([csp, ssp], axis=1)
        gv = vals * gate128
        scores = jax.lax.dot_general(
            q_feat, k_feat, (((1,), (1,)), ((), ())),
            preferred_element_type=f32)
        scores = scores * tri
        attn = dot(jnp.concatenate([scores, q_feat], axis=1),
                   jnp.concatenate([gv, state], axis=0))
        state = state + jax.lax.dot_general(
            k_feat, gv, (((0,), (0,)), ((), ())),
            preferred_element_type=f32)
        inv_g = jax.lax.rsqrt(jnp.maximum(cum_g, 1.0))
        kv_ret = attn * inv_g * (1.0 / math.sqrt(qp.shape[1]))
        kv_out = dot(kv_ret, wkv_ref[...]) + bkv_ref[...]

        comb = jnp.concatenate([pos_out, kv_out], axis=1)
        mu = jnp.mean(comb, axis=1, keepdims=True)
        dmu = comb - mu
        var = jnp.mean(dmu * dmu, axis=1, keepdims=True)
        ln = dmu * jax.lax.rsqrt(var + 1e-5) * lng_ref[...] + lnb_ref[...]
        t = _gelu(dot(ln, wt1_ref[...]) + bt1_ref[...])
        out_ref[0, sl, :] = x + dot(t, wt2_ref[...]) + bt2_ref[...]

    carry_ref[...] = carry
    state_ref[...] = state


def kernel(x, pos_phases, magnitude_scale, w_v, b_v, w_o, b_o, w_m, b_m,
           w_q, b_q, w_ke, b_ke, w_ve, b_ve, w_s1, b_s1, w_s2, b_s2,
           w_g, b_g, w_kv, b_kv, ln_g, ln_b, w_t1, b_t1, w_t2, b_t2):
    B, L, D = x.shape
    P = w_ke.shape[1]
    V = w_ve.shape[1]
    C = CHUNK
    NC = L // C

    cphi, sphi = _phase_tables(pos_phases[:L])

    ms = jnp.asarray(magnitude_scale, jnp.float32).reshape(1, 1)
    w_ve_p = jnp.pad(w_ve, ((0, 0), (0, VP - V)))
    b_ve_p = jnp.pad(b_ve.reshape(1, V), ((0, 0), (0, VP - V)))
    w_kv_p = jnp.pad(w_kv, ((0, VP - V), (0, 0)))
    w_g_p = jnp.pad(w_g, ((0, 0), (0, VP - 1)))
    b_g_p = jnp.pad(b_g.reshape(1, 1), ((0, 0), (0, VP - 1)))
    w_cat = jnp.concatenate([w_v, w_m, w_q, w_ke, w_ve_p, w_g_p], axis=1)
    b_cat = jnp.concatenate(
        [b_v.reshape(1, -1), b_m.reshape(1, -1), b_q.reshape(1, -1),
         b_ke.reshape(1, -1), b_ve_p, b_g_p], axis=1)

    row = lambda v: v.reshape(1, -1)
    full = lambda a: pl.BlockSpec(a.shape, lambda b, i: (0,) * a.ndim)

    params = [ms, w_cat, b_cat,
              w_o, row(b_o), w_s1, row(b_s1), w_s2, row(b_s2),
              w_kv_p, row(b_kv), row(ln_g), row(ln_b),
              w_t1, row(b_t1), w_t2, row(b_t2)]

    out = pl.pallas_call(
        _block_kernel,
        grid=(B, NC),
        in_specs=[
            pl.BlockSpec((1, C, D), lambda b, i: (b, i, 0)),
            pl.BlockSpec((C, D), lambda b, i: (i, 0)),
            pl.BlockSpec((C, D), lambda b, i: (i, 0)),
        ] + [full(p) for p in params],
        out_specs=pl.BlockSpec((1, C, D), lambda b, i: (b, i, 0)),
        out_shape=jax.ShapeDtypeStruct((B, L, D), jnp.float32),
        scratch_shapes=[
            pltpu.VMEM((1, 4 * D + VP), jnp.float32),
            pltpu.VMEM((2 * P, VP), jnp.float32),
        ],
        compiler_params=pltpu.CompilerParams(
            dimension_semantics=("parallel", "arbitrary"),
            vmem_limit_bytes=56 * 1024 * 1024,
        ),
        name="phasor_block_fused",
    )(x, cphi, sphi, *params)
    return out


# speedup vs baseline: 1.0233x; 1.0233x over previous
import math
from functools import partial

import jax
import jax.numpy as jnp
from jax.experimental import pallas as pl
from jax.experimental.pallas import tpu as pltpu

PI = math.pi
CHUNK = 1024
SUB = 256
VP = 128


def _gelu(z):
    return 0.5 * z * (1.0 + jax.lax.erf(z * (1.0 / math.sqrt(2.0))))


def _cossin(x):
    nf = jnp.round(x * (2.0 / math.pi))
    ni = nf.astype(jnp.int32)
    r = x - nf * 1.5707963705062866
    r = r - nf * (-4.371139000186245e-08)
    u = r * r
    c = (1.0 / 24.0) - u * (1.0 / 720.0)
    c = c * u - 0.5
    c = c * u + 1.0
    s = (1.0 / 120.0) - u * (1.0 / 5040.0)
    s = s * u - (1.0 / 6.0)
    s = s * u + 1.0
    s = s * r
    b0 = (ni & 1) != 0
    b1 = (ni & 2) != 0
    cos_base = jnp.where(b0, s, c)
    sin_base = jnp.where(b0, c, s)
    cosx = jnp.where(b0 ^ b1, -cos_base, cos_base)
    sinx = jnp.where(b1, -sin_base, sin_base)
    return cosx, sinx


def _trig_kernel(phi_ref, cos_ref, sin_ref):
    p = phi_ref[...]
    cp, sp = _cossin(p)
    cos_ref[...] = cp
    sin_ref[...] = sp


def _phase_tables(pos_phases):
    L, D = pos_phases.shape
    tc = 512
    return pl.pallas_call(
        _trig_kernel,
        grid=(L // tc,),
        in_specs=[pl.BlockSpec((tc, D), lambda i: (i, 0))],
        out_specs=[pl.BlockSpec((tc, D), lambda i: (i, 0)),
                   pl.BlockSpec((tc, D), lambda i: (i, 0))],
        out_shape=[jax.ShapeDtypeStruct((L, D), jnp.float32),
                   jax.ShapeDtypeStruct((L, D), jnp.float32)],
        compiler_params=pltpu.CompilerParams(
            dimension_semantics=("parallel",)),
        name="phasor_trig_tables",
    )(pos_phases)


def _block_kernel(x_ref, cphi_ref, sphi_ref, ms_ref,
                  wcat_ref, bcat_ref,
                  wo_ref, bo_ref, ws1_ref, bs1_ref, ws2_ref, bs2_ref,
                  wkv_ref, bkv_ref, lng_ref, lnb_ref,
                  wt1_ref, bt1_ref, wt2_ref, bt2_ref,
                  out_ref, carry_ref, state_ref):
    C = x_ref.shape[1]
    D = x_ref.shape[2]
    S = SUB
    i = pl.program_id(1)

    @pl.when(i == 0)
    def _():
        carry_ref[...] = jnp.zeros_like(carry_ref)
        state_ref[...] = jnp.zeros_like(state_ref)

    ms = jnp.abs(ms_ref[...])
    f32 = jnp.float32
    dot = partial(jnp.dot, preferred_element_type=f32)

    rows = jax.lax.broadcasted_iota(jnp.int32, (S, S), 0)
    cols = jax.lax.broadcasted_iota(jnp.int32, (S, S), 1)
    tri = (rows >= cols).astype(f32)

    carry = carry_ref[...]
    state = state_ref[...]

    for k in range(C // S):
        sl = slice(k * S, (k + 1) * S)
        x = x_ref[0, sl, :]
        cphi = cphi_ref[sl, :]
        sphi = sphi_ref[sl, :]

        z = dot(x, wcat_ref[...]) + bcat_ref[...]
        v1 = z[:, :D]
        mag = jax.nn.sigmoid(z[:, D:2 * D]) * ms
        dq = z[:, 2 * D:3 * D]
        qp = jnp.tanh(z[:, 3 * D:3 * D + VP]) * PI
        vals = z[:, 3 * D + VP:3 * D + 2 * VP]
        gate = jax.nn.sigmoid(z[:, 3 * D + 2 * VP:3 * D + 2 * VP + 1])
        wv1 = mag * v1
        gate128 = jnp.broadcast_to(gate, (S, VP))

        big = jnp.concatenate(
            [wv1 * cphi, wv1 * sphi, mag, x, gate128], axis=1)
        cums = dot(tri, big) + carry
        carry = cums[S - 1:S, :]
        cum_c = cums[:, :D]
        cum_s = cums[:, D:2 * D]
        cum_m = cums[:, 2 * D:3 * D]
        cum_x = cums[:, 3 * D:4 * D]
        cum_g = cums[:, 4 * D:]

        inv_sq = jax.lax.rsqrt(cum_m + 1e-8)
        cdq, sdq = _cossin(dq)
        cpq = cphi * cdq - sphi * sdq
        spq = sphi * cdq + cphi * sdq
        pos_ret = (cum_c * cpq + cum_s * spq) * inv_sq * (1.0 / math.sqrt(D))
        pos_out = dot(pos_ret, wo_ref[...]) + bo_ref[...]

        posf = (jax.lax.broadcasted_iota(jnp.int32, (S, 1), 0)
                + (i * C + k * S + 1)).astype(f32)
        ctx = cum_x * (1.0 / posf)
        cat = jnp.concatenate([x, ctx], axis=1)
        h = _gelu(dot(cat, ws1_ref[...]) + bs1_ref[...])
        sp = jnp.tanh(dot(h, ws2_ref[...]) + bs2_ref[...]) * PI
        cqp, sqp = _cossin(qp)
        csp, ssp = _cossin(sp)
        q_feat = jnp.concatenate([cqp, sqp], axis=1)
        k_feat = jnp.concatenate([csp, ssp], axis=1)
        gv = vals * gate128
        scores = jax.lax.dot_general(
            q_feat, k_feat, (((1,), (1,)), ((), ())),
            preferred_element_type=f32)
        scores = scores * tri
        attn = dot(jnp.concatenate([scores, q_feat], axis=1),
                   jnp.concatenate([gv, state], axis=0))
        state = state + jax.lax.dot_general(
            k_feat, gv, (((0,), (0,)), ((), ())),
            preferred_element_type=f32)
        inv_g = jax.lax.rsqrt(jnp.maximum(cum_g, 1.0))
        kv_ret = attn * inv_g * (1.0 / math.sqrt(qp.shape[1]))
        kv_out = dot(kv_ret, wkv_ref[...]) + bkv_ref[...]

        comb = jnp.concatenate([pos_out, kv_out], axis=1)
        mu = jnp.mean(comb, axis=1, keepdims=True)
        dmu = comb - mu
        var = jnp.mean(dmu * dmu, axis=1, keepdims=True)
        ln = dmu * jax.lax.rsqrt(var + 1e-5) * lng_ref[...] + lnb_ref[...]
        t = _gelu(dot(ln, wt1_ref[...]) + bt1_ref[...])
        out_ref[0, sl, :] = x + dot(t, wt2_ref[...]) + bt2_ref[...]

    carry_ref[...] = carry
    state_ref[...] = state


def kernel(x, pos_phases, magnitude_scale, w_v, b_v, w_o, b_o, w_m, b_m,
           w_q, b_q, w_ke, b_ke, w_ve, b_ve, w_s1, b_s1, w_s2, b_s2,
           w_g, b_g, w_kv, b_kv, ln_g, ln_b, w_t1, b_t1, w_t2, b_t2):
    B, L, D = x.shape
    P = w_ke.shape[1]
    V = w_ve.shape[1]
    C = CHUNK
    NC = L // C

    cphi, sphi = _phase_tables(pos_phases[:L])

    ms = jnp.asarray(magnitude_scale, jnp.float32).reshape(1, 1)
    w_ve_p = jnp.pad(w_ve, ((0, 0), (0, VP - V)))
    b_ve_p = jnp.pad(b_ve.reshape(1, V), ((0, 0), (0, VP - V)))
    w_kv_p = jnp.pad(w_kv, ((0, VP - V), (0, 0)))
    w_g_p = jnp.pad(w_g, ((0, 0), (0, VP - 1)))
    b_g_p = jnp.pad(b_g.reshape(1, 1), ((0, 0), (0, VP - 1)))
    w_cat = jnp.concatenate([w_v, w_m, w_q, w_ke, w_ve_p, w_g_p], axis=1)
    b_cat = jnp.concatenate(
        [b_v.reshape(1, -1), b_m.reshape(1, -1), b_q.reshape(1, -1),
         b_ke.reshape(1, -1), b_ve_p, b_g_p], axis=1)

    row = lambda v: v.reshape(1, -1)
    full = lambda a: pl.BlockSpec(a.shape, lambda b, i: (0,) * a.ndim)

    params = [ms, w_cat, b_cat,
              w_o, row(b_o), w_s1, row(b_s1), w_s2, row(b_s2),
              w_kv_p, row(b_kv), row(ln_g), row(ln_b),
              w_t1, row(b_t1), w_t2, row(b_t2)]

    out = pl.pallas_call(
        _block_kernel,
        grid=(B, NC),
        in_specs=[
            pl.BlockSpec((1, C, D), lambda b, i: (b, i, 0)),
            pl.BlockSpec((C, D), lambda b, i: (i, 0)),
            pl.BlockSpec((C, D), lambda b, i: (i, 0)),
        ] + [full(p) for p in params],
        out_specs=pl.BlockSpec((1, C, D), lambda b, i: (b, i, 0)),
        out_shape=jax.ShapeDtypeStruct((B, L, D), jnp.float32),
        scratch_shapes=[
            pltpu.VMEM((1, 4 * D + VP), jnp.float32),
            pltpu.VMEM((2 * P, VP), jnp.float32),
        ],
        compiler_params=pltpu.CompilerParams(
            dimension_semantics=("parallel", "arbitrary"),
            vmem_limit_bytes=56 * 1024 * 1024,
        ),
        name="phasor_block_fused",
    )(x, cphi, sphi, *params)
    return out
